# Initial kernel scaffold; baseline (speedup 1.0000x reference)
import jax
import jax.numpy as jnp
from jax.experimental import pallas as pl


def kernel(x):
    raise NotImplementedError("write your pallas kernel here")


# speedup vs baseline: 15.1688x; 15.1688x over previous
import functools

import jax
import jax.numpy as jnp
from jax.experimental import pallas as pl

_MASK_RATIO = 0.15
_ROTS = ((13, 15, 26, 6), (17, 29, 16, 24))
_K0, _K1 = 0, 42


def _threefry_bits(i_u32, shape):
    ks0 = jnp.uint32(_K0)
    ks1 = jnp.uint32(_K1)
    ks2 = jnp.uint32(_K0 ^ _K1 ^ 0x1BD11BDA)
    ks = (ks0, ks1, ks2)
    x0 = jnp.full(shape, ks0, dtype=jnp.uint32)
    x1 = i_u32 + ks1
    for d in range(5):
        for rot in _ROTS[d % 2]:
            x0 = x0 + x1
            x1 = (x1 << rot) | (x1 >> (32 - rot))
            x1 = x0 ^ x1
        x0 = x0 + ks[(d + 1) % 3]
        x1 = x1 + ks[(d + 2) % 3] + jnp.uint32(d + 1)
    return x0 ^ x1


def _mask_kernel(o_ref, *, rblk, T, num_mask):
    p = pl.program_id(0)
    r = jax.lax.broadcasted_iota(jnp.int32, (rblk, T), 0)
    t = jax.lax.broadcasted_iota(jnp.int32, (rblk, T), 1)
    i = (p * rblk + r) * T + t
    bits = _threefry_bits(i.astype(jnp.uint32), (rblk, T))
    key_u = (bits & jnp.uint32(0xFFFFFE00)) | t.astype(jnp.uint32)
    keys_s = (key_u ^ jnp.uint32(0x80000000)).astype(jnp.int32)
    prefix = jnp.zeros((rblk, 1), dtype=jnp.uint32)
    for b in range(31, -1, -1):
        cand = prefix | jnp.uint32(1 << b)
        cand_s = (cand ^ jnp.uint32(0x80000000)).astype(jnp.int32)
        cnt = jnp.sum((keys_s < cand_s).astype(jnp.int32), axis=1,
                      keepdims=True)
        prefix = jnp.where(cnt < num_mask, cand, prefix)
    theta_s = (prefix ^ jnp.uint32(0x80000000)).astype(jnp.int32)
    o_ref[:, :] = (keys_s > theta_s).astype(jnp.float32)


def kernel(x):
    B, R, T = x.shape
    rows = B * R
    num_mask = int(_MASK_RATIO * T)
    rblk = 256
    assert rows % rblk == 0 and T <= 512
    out = pl.pallas_call(
        functools.partial(_mask_kernel, rblk=rblk, T=T, num_mask=num_mask),
        out_shape=jax.ShapeDtypeStruct((rows, T), jnp.float32),
        grid=(rows // rblk,),
        out_specs=pl.BlockSpec((rblk, T), lambda p: (p, 0)),
    )()
    return out.reshape(B, R, T).astype(x.dtype)
